# Initial kernel scaffold; baseline (speedup 1.0000x reference)
import jax
import jax.numpy as jnp
from jax.experimental import pallas as pl


def kernel(inputs, embeddings):
    raise NotImplementedError("write your pallas kernel here")


# speedup vs baseline: 1.0234x; 1.0234x over previous
import functools

import jax
import jax.numpy as jnp
from jax import lax
from jax.experimental import pallas as pl
from jax.experimental.pallas import tpu as pltpu
from jax.experimental.pallas import tpu_sc as plsc

INPUT_DIM = 1000000
OUTPUT_DIM = 32
BATCH = 16384
SEQ = 50

NC = 2
NS = 16
NW = NC * NS

TOTAL = BATCH * SEQ
CHUNK = 128
N_CHUNKS = TOTAL // CHUNK
CPW = N_CHUNKS // NW


def _make_kernel():
  mesh = plsc.VectorSubcoreMesh(core_axis_name="c", subcore_axis_name="s")

  @functools.partial(
      pl.kernel,
      out_type=jax.ShapeDtypeStruct((TOTAL, OUTPUT_DIM), jnp.float32),
      mesh=mesh,
      compiler_params=pltpu.CompilerParams(use_tc_tiling_on_sc=False),
      scratch_types=[
          pltpu.VMEM((CPW, CHUNK), jnp.int32),
          pltpu.VMEM((CHUNK, OUTPUT_DIM), jnp.float32),
          pltpu.SemaphoreType.DMA,
      ],
  )
  def k(idx_hbm, table_hbm, out_hbm, idx_v, buf, gsem):
    wid = lax.axis_index("s") * NC + lax.axis_index("c")
    row0 = wid * CPW
    pltpu.sync_copy(idx_hbm.at[pl.ds(row0, CPW)], idx_v)

    def body(j, _):
      pltpu.async_copy(table_hbm.at[idx_v.at[j]], buf, gsem).wait()
      pltpu.sync_copy(buf, out_hbm.at[pl.ds((row0 + j) * CHUNK, CHUNK)])
      return _

    lax.fori_loop(0, CPW, body, None)

  return k


_lookup = _make_kernel()


@jax.jit
def kernel(inputs, embeddings):
  idx = inputs.astype(jnp.int32).reshape(N_CHUNKS, CHUNK)
  out = _lookup(idx, embeddings)
  return out.reshape(BATCH, SEQ, OUTPUT_DIM)


# speedup vs baseline: 1.1118x; 1.0864x over previous
import functools

import jax
import jax.numpy as jnp
from jax import lax
from jax.experimental import pallas as pl
from jax.experimental.pallas import tpu as pltpu
from jax.experimental.pallas import tpu_sc as plsc

INPUT_DIM = 1000000
OUTPUT_DIM = 32
BATCH = 16384
SEQ = 50

NC = 2
NS = 16
NW = NC * NS

TOTAL = BATCH * SEQ
CHUNK = 128
N_CHUNKS = TOTAL // CHUNK
CPW = N_CHUNKS // NW


NBUF = 8
LEAD = 4
NROUND = CPW // NBUF


def _make_kernel():
  mesh = plsc.VectorSubcoreMesh(core_axis_name="c", subcore_axis_name="s")

  @functools.partial(
      pl.kernel,
      out_type=jax.ShapeDtypeStruct((TOTAL, OUTPUT_DIM), jnp.float32),
      mesh=mesh,
      compiler_params=pltpu.CompilerParams(use_tc_tiling_on_sc=False),
      scratch_types=[
          pltpu.VMEM((CPW, CHUNK), jnp.int32),
          [pltpu.VMEM((CHUNK, OUTPUT_DIM), jnp.float32)] * NBUF,
          [pltpu.SemaphoreType.DMA] * NBUF,
          [pltpu.SemaphoreType.DMA] * NBUF,
      ],
  )
  def k(idx_hbm, table_hbm, out_hbm, idx_v, bufs, gsems, wsems):
    wid = lax.axis_index("s") * NC + lax.axis_index("c")
    row0 = wid * CPW
    pltpu.sync_copy(idx_hbm.at[pl.ds(row0, CPW)], idx_v)

    def gather(j, b):
      pltpu.async_copy(table_hbm.at[idx_v.at[j]], bufs[b], gsems[b])

    def gather_wait(b):
      pltpu.make_async_copy(
          table_hbm.at[idx_v.at[0]], bufs[b], gsems[b]).wait()

    def write(j, b):
      pltpu.async_copy(
          bufs[b], out_hbm.at[pl.ds((row0 + j) * CHUNK, CHUNK)], wsems[b])

    def write_wait(b):
      pltpu.make_async_copy(
          bufs[b], out_hbm.at[pl.ds(0, CHUNK)], wsems[b]).wait()

    for b in range(LEAD):
      gather(b, b)

    def round_body(g, _):
      for b in range(NBUF):
        j = g * NBUF + b
        gather_wait(b)
        write(j, b)
        bb = (b + LEAD) % NBUF
        jn = j + LEAD
        if b < NBUF - LEAD:
          @pl.when(g >= 1)
          def _():
            write_wait(bb)
          gather(jn, bb)
        else:
          write_wait(bb)
          @pl.when(g < NROUND - 1)
          def _():
            gather(jn, bb)
      return _

    lax.fori_loop(0, NROUND, round_body, None)

    for b in range(NBUF - LEAD, NBUF):
      write_wait(b)

  return k


_lookup = _make_kernel()


@jax.jit
def kernel(inputs, embeddings):
  idx = inputs.astype(jnp.int32).reshape(N_CHUNKS, CHUNK)
  out = _lookup(idx, embeddings)
  return out.reshape(BATCH, SEQ, OUTPUT_DIM)


# speedup vs baseline: 1.8051x; 1.6236x over previous
import functools

import jax
import jax.numpy as jnp
from jax import lax
from jax.experimental import pallas as pl
from jax.experimental.pallas import tpu as pltpu
from jax.experimental.pallas import tpu_sc as plsc

INPUT_DIM = 1000000
OUTPUT_DIM = 32
BATCH = 16384
SEQ = 50

NC = 2
NS = 16
NW = NC * NS

RPW = BATCH // NW
CR = 4
CPW = RPW // CR
NBUF = 8
LEAD = 4
NROUND = CPW // NBUF


def _make_kernel():
  mesh = plsc.VectorSubcoreMesh(core_axis_name="c", subcore_axis_name="s")

  @functools.partial(
      pl.kernel,
      out_type=jax.ShapeDtypeStruct((BATCH, SEQ, OUTPUT_DIM), jnp.float32),
      mesh=mesh,
      compiler_params=pltpu.CompilerParams(use_tc_tiling_on_sc=False),
      scratch_types=[
          pltpu.VMEM((RPW, SEQ), jnp.int32),
          [pltpu.VMEM((CR, SEQ, OUTPUT_DIM), jnp.float32)] * NBUF,
          [pltpu.SemaphoreType.DMA] * NBUF,
          [pltpu.SemaphoreType.DMA] * NBUF,
      ],
  )
  def k(idx_hbm, table_hbm, out_hbm, idx_v, bufs, gsems, wsems):
    wid = lax.axis_index("s") * NC + lax.axis_index("c")
    row0 = wid * RPW
    pltpu.sync_copy(idx_hbm.at[pl.ds(row0, RPW)], idx_v)

    def gather(j, b):
      for i in range(CR):
        pltpu.async_copy(
            table_hbm.at[idx_v.at[j * CR + i]], bufs[b].at[i], gsems[b])

    def gather_wait(b):
      for i in range(CR):
        pltpu.make_async_copy(
            table_hbm.at[idx_v.at[0]], bufs[b].at[i], gsems[b]).wait()

    def write(j, b):
      pltpu.async_copy(
          bufs[b], out_hbm.at[pl.ds(row0 + j * CR, CR)], wsems[b])

    def write_wait(b):
      pltpu.make_async_copy(
          bufs[b], out_hbm.at[pl.ds(0, CR)], wsems[b]).wait()

    for b in range(LEAD):
      gather(b, b)

    def round_body(g, _):
      for b in range(NBUF):
        j = g * NBUF + b
        gather_wait(b)
        write(j, b)
        bb = (b + LEAD) % NBUF
        jn = j + LEAD
        if b < NBUF - LEAD:
          @pl.when(g >= 1)
          def _():
            write_wait(bb)
          gather(jn, bb)
        else:
          write_wait(bb)
          @pl.when(g < NROUND - 1)
          def _():
            gather(jn, bb)
      return _

    lax.fori_loop(0, NROUND, round_body, None)

    for b in range(NBUF - LEAD, NBUF):
      write_wait(b)

  return k


_lookup = _make_kernel()


@jax.jit
def kernel(inputs, embeddings):
  return _lookup(inputs.astype(jnp.int32), embeddings)


# speedup vs baseline: 1.8121x; 1.0039x over previous
import functools

import jax
import jax.numpy as jnp
from jax import lax
from jax.experimental import pallas as pl
from jax.experimental.pallas import tpu as pltpu
from jax.experimental.pallas import tpu_sc as plsc

INPUT_DIM = 1000000
OUTPUT_DIM = 32
BATCH = 16384
SEQ = 50

NC = 2
NS = 16
NW = NC * NS

BPW = BATCH // NW
CHUNK = 128
BCH = BPW // CHUNK
CPW = SEQ * BCH
NBUF = 8
LEAD = 4
NROUND = CPW // NBUF


def _make_kernel():
  mesh = plsc.VectorSubcoreMesh(core_axis_name="c", subcore_axis_name="s")

  @functools.partial(
      pl.kernel,
      out_type=jax.ShapeDtypeStruct((BATCH, SEQ, OUTPUT_DIM), jnp.float32),
      mesh=mesh,
      compiler_params=pltpu.CompilerParams(use_tc_tiling_on_sc=False),
      scratch_types=[
          pltpu.VMEM((SEQ, BPW), jnp.int32),
          [pltpu.VMEM((CHUNK, OUTPUT_DIM), jnp.float32)] * NBUF,
          [pltpu.SemaphoreType.DMA] * NBUF,
          [pltpu.SemaphoreType.DMA] * NBUF,
      ],
  )
  def k(idxt_hbm, table_hbm, out_hbm, idx_v, bufs, gsems, wsems):
    wid = lax.axis_index("s") * NC + lax.axis_index("c")
    b0 = wid * BPW
    pltpu.sync_copy(idxt_hbm.at[:, pl.ds(b0, BPW)], idx_v)

    def gather(j, b):
      s = j // BCH
      bc = (j % BCH) * CHUNK
      pltpu.async_copy(
          table_hbm.at[idx_v.at[s, pl.ds(bc, CHUNK)]], bufs[b], gsems[b])

    def gather_wait(b):
      pltpu.make_async_copy(
          table_hbm.at[idx_v.at[0, pl.ds(0, CHUNK)]], bufs[b],
          gsems[b]).wait()

    def write(j, b):
      s = j // BCH
      bc = (j % BCH) * CHUNK
      pltpu.async_copy(
          bufs[b], out_hbm.at[pl.ds(b0 + bc, CHUNK), s], wsems[b])

    def write_wait(b):
      pltpu.make_async_copy(
          bufs[b], out_hbm.at[pl.ds(0, CHUNK), 0], wsems[b]).wait()

    for b in range(LEAD):
      gather(b, b)

    def round_body(g, _):
      for b in range(NBUF):
        j = g * NBUF + b
        gather_wait(b)
        write(j, b)
        bb = (b + LEAD) % NBUF
        jn = j + LEAD
        if b < NBUF - LEAD:
          @pl.when(g >= 1)
          def _():
            write_wait(bb)
          gather(jn, bb)
        else:
          write_wait(bb)
          @pl.when(g < NROUND - 1)
          def _():
            gather(jn, bb)
      return _

    lax.fori_loop(0, NROUND, round_body, None)

    for b in range(NBUF - LEAD, NBUF):
      write_wait(b)

  return k


_lookup = _make_kernel()


@jax.jit
def kernel(inputs, embeddings):
  return _lookup(inputs.astype(jnp.int32).T, embeddings)


# speedup vs baseline: 2.4549x; 1.3547x over previous
import functools

import jax
import jax.numpy as jnp
from jax import lax
from jax.experimental import pallas as pl
from jax.experimental.pallas import tpu as pltpu
from jax.experimental.pallas import tpu_sc as plsc

INPUT_DIM = 1000000
OUTPUT_DIM = 32
BATCH = 16384
SEQ = 50
SEQ_PAD = 52

NC = 2
NS = 16
NW = NC * NS

BPW = BATCH // NW
CHUNK = 128
BCH = BPW // CHUNK
CPW = SEQ * BCH
NBUF = 8
LEAD = 4
NROUND = CPW // NBUF

T1_COLS = 2048
T1_GRID = (INPUT_DIM + T1_COLS - 1) // T1_COLS
TBL_ROWS = T1_GRID * T1_COLS
T2_BATCH = 512
FPB = SEQ_PAD * OUTPUT_DIM


def _t1_body(x_ref, o_ref):
  y = x_ref[...].T
  q = T1_COLS // 4
  o_ref[...] = jnp.concatenate(
      [y[k * q:(k + 1) * q, :] for k in range(4)], axis=1)


def _table_linearize(table_t):
  return pl.pallas_call(
      _t1_body,
      grid=(T1_GRID,),
      in_specs=[pl.BlockSpec((OUTPUT_DIM, T1_COLS), lambda i: (0, i))],
      out_specs=pl.BlockSpec((T1_COLS // 4, 128), lambda i: (i, 0)),
      out_shape=jax.ShapeDtypeStruct((TBL_ROWS // 4, 128), jnp.float32),
  )(table_t)


def _t2_body(x_ref, o_ref):
  o_ref[...] = x_ref[...].T


def _out_transpose(out_flat):
  return pl.pallas_call(
      _t2_body,
      grid=(BATCH // T2_BATCH,),
      in_specs=[pl.BlockSpec((T2_BATCH, FPB), lambda i: (i, 0))],
      out_specs=pl.BlockSpec((FPB, T2_BATCH), lambda i: (0, i)),
      out_shape=jax.ShapeDtypeStruct((FPB, BATCH), jnp.float32),
  )(out_flat)


def _make_sc_kernel():
  mesh = plsc.VectorSubcoreMesh(core_axis_name="c", subcore_axis_name="s")

  @functools.partial(
      pl.kernel,
      out_type=jax.ShapeDtypeStruct((BATCH, FPB), jnp.float32),
      mesh=mesh,
      compiler_params=pltpu.CompilerParams(use_tc_tiling_on_sc=False),
      scratch_types=[
          pltpu.VMEM((SEQ, BPW), jnp.int32),
          [pltpu.VMEM((CHUNK, OUTPUT_DIM), jnp.float32)] * NBUF,
          [pltpu.SemaphoreType.DMA] * NBUF,
          [pltpu.SemaphoreType.DMA] * NBUF,
      ],
  )
  def k(idxt_hbm, table_hbm, out_hbm, idx_v, bufs, gsems, wsems):
    wid = lax.axis_index("s") * NC + lax.axis_index("c")
    b0 = wid * BPW
    pltpu.sync_copy(idxt_hbm.at[:, pl.ds(b0, BPW)], idx_v)

    def gather(j, b):
      s = j // BCH
      bc = (j % BCH) * CHUNK
      pltpu.async_copy(
          table_hbm.at[idx_v.at[s, pl.ds(bc, CHUNK)]], bufs[b], gsems[b])

    def gather_wait(b):
      pltpu.make_async_copy(
          table_hbm.at[idx_v.at[0, pl.ds(0, CHUNK)]], bufs[b],
          gsems[b]).wait()

    def write(j, b):
      s = j // BCH
      bc = (j % BCH) * CHUNK
      pltpu.async_copy(
          bufs[b],
          out_hbm.at[pl.ds(b0 + bc, CHUNK), pl.ds(s * OUTPUT_DIM,
                                                  OUTPUT_DIM)],
          wsems[b])

    def write_wait(b):
      pltpu.make_async_copy(
          bufs[b], out_hbm.at[pl.ds(0, CHUNK), pl.ds(0, OUTPUT_DIM)],
          wsems[b]).wait()

    for b in range(LEAD):
      gather(b, b)

    def round_body(g, _):
      for b in range(NBUF):
        j = g * NBUF + b
        gather_wait(b)
        write(j, b)
        bb = (b + LEAD) % NBUF
        jn = j + LEAD
        if b < NBUF - LEAD:
          @pl.when(g >= 1)
          def _():
            write_wait(bb)
          gather(jn, bb)
        else:
          write_wait(bb)
          @pl.when(g < NROUND - 1)
          def _():
            gather(jn, bb)
      return _

    lax.fori_loop(0, NROUND, round_body, None)

    for b in range(NBUF - LEAD, NBUF):
      write_wait(b)

  return k


_sc_gather = _make_sc_kernel()


@jax.jit
def kernel(inputs, embeddings):
  table_lin = _table_linearize(embeddings.T)
  tbl = table_lin.reshape(TBL_ROWS, OUTPUT_DIM)
  t = inputs.astype(jnp.int32)
  idx_s = ((t >> 11) << 11) | ((t & 511) << 2) | ((t >> 9) & 3)
  out = _sc_gather(idx_s.T, tbl)
  out_t = _out_transpose(out)
  return (out_t[:SEQ * OUTPUT_DIM]
          .reshape(SEQ, OUTPUT_DIM, BATCH).transpose(2, 0, 1))
